# baseline (speedup 1.0000x reference)
import functools

import jax
import jax.numpy as jnp
from jax import lax
from jax.experimental import pallas as pl
from jax.experimental.pallas import tpu as pltpu
from jax.experimental.pallas import tpu_sc as plsc

B = 16384
D = 32
NC = 2
NS = 16
L = 16
NW = NC * NS
BPW = B // NW
CHUNK = 128
NCH = BPW // CHUNK

N_U = 1000000
N_M = 100000
BC = 32768

CC = 1024
TPC = CC // 128
R_SC = 491520
CPT = R_SC // NW
NCHUNK = CPT // CC
BLK_OFF = R_SC // BC

_mesh = plsc.VectorSubcoreMesh(core_axis_name="c", subcore_axis_name="s")


def _mv_body(x_ref, w_ref, b_ref, o_ref):
    o_ref[...] = jnp.sum(x_ref[...] * w_ref[...], axis=0) + b_ref[0, 0]


def _matvec(table_t, w, bias, out_n, blk_off):
    return pl.pallas_call(
        _mv_body,
        grid=(pl.cdiv(out_n, BC),),
        in_specs=[
            pl.BlockSpec((D, BC), lambda i, o=blk_off: (0, i + o)),
            pl.BlockSpec((D, 1), lambda i: (0, 0)),
            pl.BlockSpec((1, 1), lambda i: (0, 0)),
        ],
        out_specs=pl.BlockSpec((BC,), lambda i: (i,)),
        out_shape=jax.ShapeDtypeStruct((out_n,), jnp.float32),
    )(table_t, w, bias)


@functools.partial(
    pl.kernel,
    out_type=jax.ShapeDtypeStruct((R_SC,), jnp.float32),
    mesh=_mesh,
    compiler_params=pltpu.CompilerParams(
        needs_layout_passes=False, use_tc_tiling_on_sc=True),
    scratch_types=[
        pltpu.VMEM((2, 4, 8, CC), jnp.float32),
        pltpu.VMEM((CC,), jnp.float32),
        pltpu.VMEM((48,), jnp.float32),
        pltpu.SemaphoreType.DMA,
        pltpu.SemaphoreType.DMA,
    ],
)
def _sc_scan(ut_hbm, wb_hbm, out_hbm, slab_v, sco_v, wb_v, sem0, sem1):
    wid = lax.axis_index("s") * NC + lax.axis_index("c")
    base = wid * CPT
    pltpu.sync_copy(wb_hbm, wb_v)
    w0 = wb_v[pl.ds(0, L)]
    w1 = wb_v[pl.ds(L, L)]
    bias = wb_v[pl.ds(2 * L, L)][0]
    sems = (sem0, sem1)

    def fire(ci):
        buf, c0 = ci % 2, base + ci * CC
        return [
            pltpu.async_copy(
                ut_hbm.at[pl.ds(8 * k, 8), pl.ds(c0, CC)],
                slab_v.at[buf, k], sems[buf])
            for k in range(4)
        ]

    def compute_store(ci):
        buf, c0 = ci % 2, base + ci * CC

        def grp_body(g, _):
            sl = pl.ds(g * L, L)
            acc = jnp.full((L,), bias, jnp.float32)
            for d in range(D):
                v = slab_v[buf, d // 8, d % 8, sl]
                wv = w0 if d < L else w1
                acc = acc + v * wv[d % L]
            sco_v[sl] = acc
            return 0

        lax.fori_loop(0, CC // L, grp_body, 0)
        pltpu.sync_copy(sco_v, out_hbm.at[pl.ds(c0, CC)])

    pending = {0: fire(0)}
    for ci in range(NCHUNK):
        if ci + 1 < NCHUNK:
            pending[ci + 1] = fire(ci + 1)
        for h in pending.pop(ci):
            h.wait()
        compute_store(ci)


@functools.partial(
    pl.kernel,
    out_type=jax.ShapeDtypeStruct((B,), jnp.float32),
    mesh=_mesh,
    compiler_params=pltpu.CompilerParams(
        needs_layout_passes=False, use_tc_tiling_on_sc=False),
    scratch_types=[
        pltpu.VMEM((NCH, CHUNK), jnp.int32),
        pltpu.VMEM((NCH, CHUNK), jnp.int32),
        pltpu.VMEM((BPW,), jnp.float32),
        pltpu.VMEM((BPW,), jnp.float32),
        pltpu.VMEM((BPW,), jnp.float32),
        pltpu.SemaphoreType.DMA,
    ],
)
def _sc_gather_add(users_hbm, movies_hbm, su_hbm, sm_hbm, out_hbm,
                   uidx_v, midx_v, su_v, sm_v, out_v, sem):
    wid = lax.axis_index("s") * NC + lax.axis_index("c")
    pltpu.sync_copy(users_hbm.at[pl.ds(wid * NCH, NCH)], uidx_v)
    pltpu.sync_copy(movies_hbm.at[pl.ds(wid * NCH, NCH)], midx_v)
    copies = []
    for j in range(NCH):
        copies.append(pltpu.async_copy(
            su_hbm.at[uidx_v.at[j]], su_v.at[pl.ds(j * CHUNK, CHUNK)], sem))
        copies.append(pltpu.async_copy(
            sm_hbm.at[midx_v.at[j]], sm_v.at[pl.ds(j * CHUNK, CHUNK)], sem))
    for c in copies:
        c.wait()
    for i in range(BPW // L):
        sl = pl.ds(i * L, L)
        out_v[sl] = su_v[sl] + sm_v[sl]
    pltpu.sync_copy(out_v, out_hbm.at[pl.ds(wid * BPW, BPW)])


def kernel(users, movies, user_table, movie_table, W, b):
    wf = W.reshape(-1).astype(jnp.float32)
    wu = wf[:D].reshape(D, 1)
    wm = wf[D:].reshape(D, 1)
    bias = b.astype(jnp.float32).reshape(1, 1)
    zero = jnp.zeros((1, 1), jnp.float32)
    wb_u = jnp.concatenate(
        [wf[:D], b.astype(jnp.float32), jnp.zeros((15,), jnp.float32)])
    ut_t = user_table.T
    su_sc = _sc_scan(ut_t, wb_u)
    su_tc = _matvec(ut_t, wu, bias, N_U - R_SC, BLK_OFF)
    sm = _matvec(movie_table.T, wm, zero, N_M, 0)
    su = jnp.concatenate([su_sc, su_tc])
    u2 = users.astype(jnp.int32).reshape(NW * NCH, CHUNK)
    m2 = movies.astype(jnp.int32).reshape(NW * NCH, CHUNK)
    out = _sc_gather_add(u2, m2, su, sm)
    return out.reshape(B, 1)


# speedup vs baseline: 1.1503x; 1.1503x over previous
import functools

import jax
import jax.numpy as jnp
from jax import lax
from jax.experimental import pallas as pl
from jax.experimental.pallas import tpu as pltpu
from jax.experimental.pallas import tpu_sc as plsc

B = 16384
D = 32
NC = 2
NS = 16
L = 16
NW = NC * NS
BPW = B // NW
CHUNK = 128
NCH = BPW // CHUNK

BC = 65536


def _mv_body(x_ref, w_ref, b_ref, o_ref):
    o_ref[...] = jnp.sum(x_ref[...] * w_ref[...], axis=0) + b_ref[0, 0]


def _matvec(table_t, w, bias):
    n = table_t.shape[1]
    grid = pl.cdiv(n, BC)
    return pl.pallas_call(
        _mv_body,
        grid=(grid,),
        in_specs=[
            pl.BlockSpec((D, BC), lambda i: (0, i)),
            pl.BlockSpec((D, 1), lambda i: (0, 0)),
            pl.BlockSpec((1, 1), lambda i: (0, 0)),
        ],
        out_specs=pl.BlockSpec((BC,), lambda i: (i,)),
        out_shape=jax.ShapeDtypeStruct((n,), jnp.float32),
    )(table_t, w, bias)


_mesh = plsc.VectorSubcoreMesh(core_axis_name="c", subcore_axis_name="s")


@functools.partial(
    pl.kernel,
    out_type=jax.ShapeDtypeStruct((B,), jnp.float32),
    mesh=_mesh,
    compiler_params=pltpu.CompilerParams(
        needs_layout_passes=False, use_tc_tiling_on_sc=False),
    scratch_types=[
        pltpu.VMEM((NCH, CHUNK), jnp.int32),
        pltpu.VMEM((NCH, CHUNK), jnp.int32),
        pltpu.VMEM((BPW,), jnp.float32),
        pltpu.VMEM((BPW,), jnp.float32),
        pltpu.VMEM((BPW,), jnp.float32),
        pltpu.SemaphoreType.DMA,
    ],
)
def _sc_gather_add(users_hbm, movies_hbm, su_hbm, sm_hbm, out_hbm,
                   uidx_v, midx_v, su_v, sm_v, out_v, sem):
    wid = lax.axis_index("s") * NC + lax.axis_index("c")
    pltpu.sync_copy(users_hbm.at[pl.ds(wid * NCH, NCH)], uidx_v)
    pltpu.sync_copy(movies_hbm.at[pl.ds(wid * NCH, NCH)], midx_v)
    copies = []
    for j in range(NCH):
        copies.append(pltpu.async_copy(
            su_hbm.at[uidx_v.at[j]], su_v.at[pl.ds(j * CHUNK, CHUNK)], sem))
        copies.append(pltpu.async_copy(
            sm_hbm.at[midx_v.at[j]], sm_v.at[pl.ds(j * CHUNK, CHUNK)], sem))
    for c in copies:
        c.wait()
    for i in range(BPW // L):
        sl = pl.ds(i * L, L)
        out_v[sl] = su_v[sl] + sm_v[sl]
    pltpu.sync_copy(out_v, out_hbm.at[pl.ds(wid * BPW, BPW)])


def kernel(users, movies, user_table, movie_table, W, b):
    wf = W.reshape(-1).astype(jnp.float32)
    wu = wf[:D].reshape(D, 1)
    wm = wf[D:].reshape(D, 1)
    bias = b.astype(jnp.float32).reshape(1, 1)
    zero = jnp.zeros((1, 1), jnp.float32)
    sm = _matvec(movie_table.T, wm, zero)
    su = _matvec(user_table.T, wu, bias)
    u2 = users.astype(jnp.int32).reshape(NW * NCH, CHUNK)
    m2 = movies.astype(jnp.int32).reshape(NW * NCH, CHUNK)
    out = _sc_gather_add(u2, m2, su, sm)
    return out.reshape(B, 1)
